# baseline (device time: 112216 ns/iter reference)
import jax
import jax.numpy as jnp
from jax import lax
from jax.experimental import pallas as pl
from jax.experimental.pallas import tpu as pltpu

N_DEV = 8
B, SQ, SKV, DH = 2, 512, 512, 64
H_PER = 8
ROWS = B * SQ
CHUNK = ROWS // N_DEV
DM = 768


def kernel(x, Wq, K_ext, V_ext, Wo):
    my = lax.axis_index("i")
    Kh = lax.dynamic_slice_in_dim(K_ext, my * H_PER, H_PER, axis=2)
    Vh = lax.dynamic_slice_in_dim(V_ext, my * H_PER, H_PER, axis=2)

    def body(x_ref, wq_ref, k_ref, v_ref, wo_ref, out_ref,
             acc_ref, rs_buf, rs_send, rs_recv, ag_send, ag_recv):
        my_pos = lax.axis_index("i")
        left = (my_pos - 1) % N_DEV
        right = (my_pos + 1) % N_DEV

        x2 = x_ref[...].reshape(ROWS, DM)
        Q = jnp.dot(x2, wq_ref[...], preferred_element_type=jnp.float32)

        rb = lax.broadcasted_iota(jnp.int32, (SQ, SKV), 0) // 64
        cb = lax.broadcasted_iota(jnp.int32, (SQ, SKV), 1) // 64
        mask = (rb == cb) | (cb == 0) | ((rb + cb) % 3 == 0)

        ctx_rows = []
        for b in range(B):
            ctx_heads = []
            for h in range(H_PER):
                q = Q[b * SQ:(b + 1) * SQ, h * DH:(h + 1) * DH]
                k = k_ref[b, :, h, :]
                v = v_ref[b, :, h, :]
                s = lax.dot_general(
                    q, k, (((1,), (1,)), ((), ())),
                    preferred_element_type=jnp.float32,
                ) * 0.125
                s = jnp.where(mask, s, -1e9)
                m = jnp.max(s, axis=1, keepdims=True)
                e = jnp.exp(s - m)
                w = e / jnp.sum(e, axis=1, keepdims=True)
                ctx_heads.append(
                    jnp.dot(w, v, preferred_element_type=jnp.float32))
            ctx_rows.append(jnp.concatenate(ctx_heads, axis=1))
        ctx = jnp.concatenate(ctx_rows, axis=0)

        partial = jnp.dot(ctx, wo_ref[...], preferred_element_type=jnp.float32)
        acc_ref[...] = partial.reshape(N_DEV, CHUNK, DM)

        barrier = pltpu.get_barrier_semaphore()
        for nbr in (left, right):
            pl.semaphore_signal(
                barrier, inc=1,
                device_id=(nbr,), device_id_type=pl.DeviceIdType.MESH)
        pl.semaphore_wait(barrier, 2)

        for h in range(N_DEV - 1):
            send_c = (my_pos - h) % N_DEV
            recv_c = (my_pos - h - 1) % N_DEV
            rdma = pltpu.make_async_remote_copy(
                src_ref=acc_ref.at[send_c],
                dst_ref=rs_buf.at[h],
                send_sem=rs_send.at[h],
                recv_sem=rs_recv.at[h],
                device_id=(right,),
                device_id_type=pl.DeviceIdType.MESH,
            )
            rdma.start()
            rdma.wait()
            acc_ref[recv_c] = acc_ref[recv_c] + rs_buf[h]

        for h in range(N_DEV - 1):
            send_c = (my_pos + 1 - h) % N_DEV
            rdma = pltpu.make_async_remote_copy(
                src_ref=acc_ref.at[send_c],
                dst_ref=acc_ref.at[send_c],
                send_sem=ag_send.at[h],
                recv_sem=ag_recv.at[h],
                device_id=(right,),
                device_id_type=pl.DeviceIdType.MESH,
            )
            rdma.start()
            rdma.wait()

        out_ref[...] = acc_ref[...].reshape(B, SQ, DM)

    return pl.pallas_call(
        body,
        out_shape=jax.ShapeDtypeStruct((B, SQ, DM), jnp.float32),
        in_specs=[pl.BlockSpec(memory_space=pltpu.VMEM)] * 5,
        out_specs=pl.BlockSpec(memory_space=pltpu.VMEM),
        scratch_shapes=[
            pltpu.VMEM((N_DEV, CHUNK, DM), jnp.float32),
            pltpu.VMEM((N_DEV - 1, CHUNK, DM), jnp.float32),
            pltpu.SemaphoreType.DMA((N_DEV - 1,)),
            pltpu.SemaphoreType.DMA((N_DEV - 1,)),
            pltpu.SemaphoreType.DMA((N_DEV - 1,)),
            pltpu.SemaphoreType.DMA((N_DEV - 1,)),
        ],
        compiler_params=pltpu.CompilerParams(collective_id=0),
    )(x, Wq, Kh, Vh, Wo)
